# baseline (speedup 1.0000x reference)
import functools

import jax
import jax.numpy as jnp
from jax import lax
from jax.experimental import pallas as pl
from jax.experimental.pallas import tpu as pltpu
from jax.experimental.pallas import tpu_sc as plsc

H = 128
ND, NI, NT = 124, 177, 104
KD, KI, KT = NI * NT, ND * NT, NI * ND
KB = 4096
NKD, NKI, NKT = pl.cdiv(KD, KB), pl.cdiv(KI, KB), pl.cdiv(KT, KB)
PROJ = NKD + NKI + NKT
DC = 4
NZ = ND // DC
F32 = jnp.float32

ED, EI, ET = 2000, 3000, 1500
EPAD_D, EPAD_I, EPAD_T = 2048, 3072, 1536
CPT_D, CPT_I, CPT_T = 128, 192, 96
SPAD_D, SPAD_I, SPAD_T = 15488, 31360, 10880
ZSL_D, ZSL_I, ZSL_T = 968, 1960, 680
OWNER_D, OWNER_I, OWNER_T = 1, 0, 1


def _sc_adjacency(ed_flat, ei_flat, et_flat):
    mesh = plsc.VectorSubcoreMesh(core_axis_name="c", subcore_axis_name="s")

    @functools.partial(
        pl.kernel, mesh=mesh,
        out_type=[jax.ShapeDtypeStruct((SPAD_D,), F32),
                  jax.ShapeDtypeStruct((SPAD_I,), F32),
                  jax.ShapeDtypeStruct((SPAD_T,), F32)],
        scratch_types=[
            pltpu.VMEM((CPT_D,), jnp.int32), pltpu.VMEM((CPT_D,), jnp.int32),
            pltpu.VMEM((CPT_I,), jnp.int32), pltpu.VMEM((CPT_I,), jnp.int32),
            pltpu.VMEM((CPT_T,), jnp.int32), pltpu.VMEM((CPT_T,), jnp.int32),
            pltpu.VMEM((CPT_D,), jnp.int32), pltpu.VMEM((CPT_D,), F32),
            pltpu.VMEM((96,), jnp.int32), pltpu.VMEM((96,), F32),
            pltpu.VMEM((96,), jnp.int32), pltpu.VMEM((96,), F32),
            pltpu.VMEM((CPT_T,), jnp.int32), pltpu.VMEM((CPT_T,), F32),
            pltpu.VMEM((1968,), F32),
            pltpu.VMEM_SHARED((SPAD_D,), F32),
            pltpu.VMEM_SHARED((SPAD_I,), F32),
            pltpu.VMEM_SHARED((SPAD_T,), F32),
        ],
    )
    def adj(ed_hbm, ei_hbm, et_hbm, outd, outi, outt,
            srcd, dstd, srci, dsti, srct, dstt,
            idxd, vald, idxia, valia, idxib, valib, idxt, valt,
            zbuf, shd, shi, sht):
        c = lax.axis_index("c")
        s = lax.axis_index("s")
        iota = lax.broadcasted_iota(jnp.int32, (16,), 0)

        def zb(j, _):
            zbuf[pl.ds(j * 16, 16)] = jnp.zeros((16,), F32)
            return 0
        lax.fori_loop(0, 1968 // 16, zb, 0)

        def build(e_hbm, epad, cpt, e_real, n, src_r, dst_r, idx_rs, val_rs):
            lo = s * cpt
            pltpu.sync_copy(e_hbm.at[pl.ds(lo, cpt)], src_r)
            pltpu.sync_copy(e_hbm.at[pl.ds(epad + lo, cpt)], dst_r)
            seg = cpt // len(idx_rs)
            for j in range(cpt // 16):
                sv = src_r[pl.ds(j * 16, 16)]
                dv = dst_r[pl.ds(j * 16, 16)]
                ok = (lo + j * 16 + iota) < e_real
                r, o = divmod(j * 16, seg)
                idx_rs[r][pl.ds(o, 16)] = dv * n + sv
                val_rs[r][pl.ds(o, 16)] = jnp.where(ok, 1.0, 0.0)

        @pl.when(c == OWNER_D)
        def _():
            build(ed_hbm, EPAD_D, CPT_D, ED, ND, srcd, dstd, [idxd], [vald])
            pltpu.sync_copy(zbuf.at[pl.ds(0, ZSL_D)],
                            shd.at[pl.ds(s * ZSL_D, ZSL_D)])
            build(et_hbm, EPAD_T, CPT_T, ET, NT, srct, dstt, [idxt], [valt])
            pltpu.sync_copy(zbuf.at[pl.ds(0, ZSL_T)],
                            sht.at[pl.ds(s * ZSL_T, ZSL_T)])

        @pl.when(c == OWNER_I)
        def _():
            build(ei_hbm, EPAD_I, CPT_I, EI, NI, srci, dsti,
                  [idxia, idxib], [valia, valib])
            pltpu.sync_copy(zbuf.at[pl.ds(0, ZSL_I)],
                            shi.at[pl.ds(s * ZSL_I, ZSL_I)])

        plsc.subcore_barrier()

        @pl.when(c == OWNER_D)
        def _():
            pltpu.sync_copy(vald, shd.at[idxd], add=True)
            pltpu.sync_copy(valt, sht.at[idxt], add=True)

        @pl.when(c == OWNER_I)
        def _():
            pltpu.sync_copy(valia, shi.at[idxia], add=True)
            pltpu.sync_copy(valib, shi.at[idxib], add=True)

        plsc.subcore_barrier()

        @pl.when(c == OWNER_D)
        def _():
            pltpu.sync_copy(shd.at[pl.ds(s * ZSL_D, ZSL_D)],
                            zbuf.at[pl.ds(0, ZSL_D)])
            pltpu.sync_copy(zbuf.at[pl.ds(0, ZSL_D)],
                            outd.at[pl.ds(s * ZSL_D, ZSL_D)])
            pltpu.sync_copy(sht.at[pl.ds(s * ZSL_T, ZSL_T)],
                            zbuf.at[pl.ds(0, ZSL_T)])
            pltpu.sync_copy(zbuf.at[pl.ds(0, ZSL_T)],
                            outt.at[pl.ds(s * ZSL_T, ZSL_T)])

        @pl.when(c == OWNER_I)
        def _():
            pltpu.sync_copy(shi.at[pl.ds(s * ZSL_I, ZSL_I)],
                            zbuf.at[pl.ds(0, ZSL_I)])
            pltpu.sync_copy(zbuf.at[pl.ds(0, ZSL_I)],
                            outi.at[pl.ds(s * ZSL_I, ZSL_I)])

    return adj(ed_flat, ei_flat, et_flat)


def _masked_acc(h_ref, x_ref, w_ref, lk, nk, ktot):
    @pl.when(lk < nk - 1)
    def _():
        h_ref[...] += jnp.dot(x_ref[...], w_ref[...],
                              preferred_element_type=F32)

    @pl.when(lk == nk - 1)
    def _():
        xb = x_ref[...]
        wb = w_ref[...]
        valid = ktot - (nk - 1) * KB
        col = lax.broadcasted_iota(jnp.int32, xb.shape, 1)
        xb = jnp.where(col < valid, xb, 0.0)
        row = lax.broadcasted_iota(jnp.int32, wb.shape, 0)
        wb = jnp.where(row < valid, wb, 0.0)
        h_ref[...] += jnp.dot(xb, wb, preferred_element_type=F32)


def _branch_dense(h0, a, n, w1, b1, w2, b2):
    eye = (lax.broadcasted_iota(jnp.int32, (n, n), 0)
           == lax.broadcasted_iota(jnp.int32, (n, n), 1)).astype(F32)
    ahat = a + eye
    deg = jnp.sum(ahat, axis=1, keepdims=True)
    dinv = lax.rsqrt(deg)

    def gcn(h, w, bias):
        p = jnp.dot(h, w, preferred_element_type=F32)
        return dinv * jnp.dot(ahat, dinv * p, preferred_element_type=F32,
                              precision=lax.Precision.HIGHEST) + bias

    h1 = jnp.maximum(gcn(h0, w1, b1), 0.0)
    return gcn(h1, w2, b2)


def _body(xd_ref, wd_ref, bd_ref, xi_ref, wi_ref, bi_ref,
          xt_ref, wt_ref, bt_ref, ad_ref, ai_ref, at_ref,
          w1d_ref, b1d_ref, w2d_ref, b2d_ref,
          w1i_ref, b1i_ref, w2i_ref, b2i_ref,
          w1t_ref, b1t_ref, w2t_ref, b2t_ref,
          o_ref, hd, hi, ht, dxs, ixs, txs):
    k = pl.program_id(0)

    @pl.when(k == 0)
    def _():
        hd[...] = jnp.broadcast_to(bd_ref[...], hd.shape)

    @pl.when(k < NKD)
    def _():
        _masked_acc(hd, xd_ref, wd_ref, k, NKD, KD)

    @pl.when(k == NKD)
    def _():
        hi[...] = jnp.broadcast_to(bi_ref[...], hi.shape)
        dxs[...] = _branch_dense(hd[...], ad_ref[...], ND,
                                 w1d_ref[...], b1d_ref[...],
                                 w2d_ref[...], b2d_ref[...])

    @pl.when((k >= NKD) & (k < NKD + NKI))
    def _():
        _masked_acc(hi, xi_ref, wi_ref, k - NKD, NKI, KI)

    @pl.when(k == NKD + NKI)
    def _():
        ht[...] = jnp.broadcast_to(bt_ref[...], ht.shape)
        ixs[...] = _branch_dense(hi[...], ai_ref[...], NI,
                                 w1i_ref[...], b1i_ref[...],
                                 w2i_ref[...], b2i_ref[...])

    @pl.when((k >= NKD + NKI) & (k < PROJ))
    def _():
        _masked_acc(ht, xt_ref, wt_ref, k - NKD - NKI, NKT, KT)

    @pl.when(k == PROJ)
    def _():
        txs[...] = _branch_dense(ht[...], at_ref[...], NT,
                                 w1t_ref[...], b1t_ref[...],
                                 w2t_ref[...], b2t_ref[...])

    @pl.when(k >= PROJ)
    def _():
        b = k - PROJ
        ixv = ixs[...]
        txv = txs[...]
        for c in range(DC):
            dxr = dxs[pl.ds(b * DC + c, 1), :]
            t = txv * dxr
            zi = lax.dot_general(ixv, t, (((1,), (1,)), ((), ())),
                                 preferred_element_type=F32)
            o_ref[pl.ds(c, 1)] = jnp.maximum(zi, 0.0)[None]


def kernel(drug_graph, drug_x, disease_graph, disease_x, target_graph, target_x,
           Wp_d, bp_d, W1_d, b1_d, W2_d, b2_d,
           Wp_i, bp_i, W1_i, b1_i, W2_i, b2_i,
           Wp_t, bp_t, W1_t, b1_t, W2_t, b2_t):
    def flat_pad(e, epad):
        e = jnp.asarray(e, jnp.int32)
        return jnp.pad(e, ((0, 0), (0, epad - e.shape[1]))).reshape(2 * epad)

    ad_f, ai_f, at_f = _sc_adjacency(flat_pad(drug_graph, EPAD_D),
                                     flat_pad(disease_graph, EPAD_I),
                                     flat_pad(target_graph, EPAD_T))
    a_d = ad_f[:ND * ND].reshape(ND, ND)
    a_i = ai_f[:NI * NI].reshape(NI, NI)
    a_t = at_f[:NT * NT].reshape(NT, NT)

    c0 = lambda k: (0, 0)
    specs = [
        pl.BlockSpec((ND, KB), lambda k: (0, jnp.clip(k, 0, NKD - 1))),
        pl.BlockSpec((KB, H), lambda k: (jnp.clip(k, 0, NKD - 1), 0)),
        pl.BlockSpec((1, H), c0),
        pl.BlockSpec((NI, KB), lambda k: (0, jnp.clip(k - NKD, 0, NKI - 1))),
        pl.BlockSpec((KB, H), lambda k: (jnp.clip(k - NKD, 0, NKI - 1), 0)),
        pl.BlockSpec((1, H), c0),
        pl.BlockSpec((NT, KB), lambda k: (0, jnp.clip(k - NKD - NKI, 0, NKT - 1))),
        pl.BlockSpec((KB, H), lambda k: (jnp.clip(k - NKD - NKI, 0, NKT - 1), 0)),
        pl.BlockSpec((1, H), c0),
    ]
    full = lambda s: pl.BlockSpec(s, lambda k: tuple(0 for _ in s))
    specs += [full((ND, ND)), full((NI, NI)), full((NT, NT))]
    small = []
    for w, bias in ((W1_d, b1_d), (W2_d, b2_d), (W1_i, b1_i),
                    (W2_i, b2_i), (W1_t, b1_t), (W2_t, b2_t)):
        small += [w, bias.reshape(1, H)]
        specs += [full((H, H)), full((1, H))]

    return pl.pallas_call(
        _body,
        grid=(PROJ + NZ,),
        in_specs=specs,
        out_specs=pl.BlockSpec(
            (DC, NI, NT), lambda k: (jnp.clip(k - PROJ, 0, NZ - 1), 0, 0)),
        out_shape=jax.ShapeDtypeStruct((ND, NI, NT), F32),
        scratch_shapes=[pltpu.VMEM((ND, H), F32), pltpu.VMEM((NI, H), F32),
                        pltpu.VMEM((NT, H), F32), pltpu.VMEM((ND, H), F32),
                        pltpu.VMEM((NI, H), F32), pltpu.VMEM((NT, H), F32)],
    )(drug_x, Wp_d, bp_d.reshape(1, H), disease_x, Wp_i, bp_i.reshape(1, H),
      target_x, Wp_t, bp_t.reshape(1, H), a_d, a_i, a_t, *small)
